# baseline (speedup 1.0000x reference)
import jax
import jax.numpy as jnp
import numpy as np
from jax.experimental import pallas as pl
from jax.experimental.pallas import tpu as pltpu

_SUB = 256


def _mlp_kernel(g1_ref, g2_ref, g3_ref, b_ref, x_ref, o_ref):
    g1 = g1_ref[...]
    g2 = g2_ref[...]
    g3 = g3_ref[...]
    b1 = b_ref[0:1, 0:512]
    b2 = b_ref[1:2, 0:512]
    b3 = b_ref[2:3, 0:32]
    tr = x_ref.shape[0]
    sub = _SUB if tr % _SUB == 0 else tr
    for s in range(tr // sub):
        rows = pl.ds(s * sub, sub)
        x = x_ref[rows, :]
        h1 = jnp.dot(x, g1, preferred_element_type=jnp.float32)
        h1 = jnp.maximum(h1 + b1, 0.0)
        h2a = jnp.dot(h1[:, 0:256], g2, preferred_element_type=jnp.float32)
        h2b = jnp.dot(h1[:, 256:512], g2, preferred_element_type=jnp.float32)
        h2 = jnp.concatenate([h2a, h2b], axis=1)
        h2 = jnp.maximum(h2 + b2, 0.0)
        y = jnp.dot(h2, g3, preferred_element_type=jnp.float32)
        o_ref[rows, :] = y + b3


_M1 = np.asarray(
    np.arange(128)[:, None] // 4 == np.arange(512)[None, :] // 16, np.float32)
_M2 = np.asarray(
    np.arange(256)[:, None] // 16 == np.arange(256)[None, :] // 16, np.float32)
_M3 = np.asarray(
    np.arange(512)[:, None] // 16 == np.arange(32)[None, :], np.float32)


def _build_mats(packed):
    w1 = packed[0:16, 0:4].T
    b1 = packed[0:16, 4]
    w2 = packed[16:32, 0:16].T
    b2 = packed[16:32, 16]
    w3 = packed[0, 64:80]
    b3 = packed[0, 80]
    g1 = jnp.tile(w1, (32, 32)) * _M1
    g2 = jnp.tile(w2, (16, 16)) * _M2
    g3 = jnp.tile(w3.reshape(16, 1), (32, 32)) * _M3
    b = jnp.stack([jnp.tile(b1, 32), jnp.tile(b2, 32),
                   jnp.full((512,), b3, jnp.float32)])
    return g1, g2, g3, b


@jax.jit
def _forward(x, packed):
    bsz = x.shape[0]
    rows = bsz // 32
    tr = 2048 if rows % 2048 == 0 else rows
    x2 = x.reshape(rows, 128)
    g1, g2, g3, b = _build_mats(packed)
    out = pl.pallas_call(
        _mlp_kernel,
        out_shape=jax.ShapeDtypeStruct((rows, 32), jnp.float32),
        grid=(rows // tr,),
        in_specs=[
            pl.BlockSpec((128, 512), lambda i: (0, 0)),
            pl.BlockSpec((256, 256), lambda i: (0, 0)),
            pl.BlockSpec((512, 32), lambda i: (0, 0)),
            pl.BlockSpec((3, 512), lambda i: (0, 0)),
            pl.BlockSpec((tr, 128), lambda i: (i, 0)),
        ],
        out_specs=pl.BlockSpec((tr, 32), lambda i: (i, 0)),
        compiler_params=pltpu.CompilerParams(
            dimension_semantics=("parallel",),
            vmem_limit_bytes=64 * 1024 * 1024,
        ),
    )(g1, g2, g3, b, x2)
    return out.reshape(bsz, 1)


def kernel(x, packed):
    return _forward(x, packed)


# speedup vs baseline: 29.6848x; 29.6848x over previous
import jax
import jax.numpy as jnp
from jax.experimental import pallas as pl
from jax.experimental.pallas import tpu as pltpu

_TB = 32768
_CHUNK = 8192


def _mlp_kernel(p_ref, x_ref, o_ref):
    w1 = p_ref[0:16, 0:4].astype(jnp.bfloat16)
    b1 = p_ref[0:16, 4:5]
    w2 = p_ref[16:32, 0:16].astype(jnp.bfloat16)
    b2 = p_ref[16:32, 16:17]
    w3 = p_ref[0:1, 64:80].astype(jnp.bfloat16)
    b3 = p_ref[0:1, 80:81]

    tb = x_ref.shape[1]
    chunk = _CHUNK if tb % _CHUNK == 0 else tb
    for c in range(tb // chunk):
        lanes = pl.ds(c * chunk, chunk)
        x = x_ref[:, lanes].astype(jnp.bfloat16)
        h = jnp.dot(w1, x, preferred_element_type=jnp.float32)
        h = jnp.maximum(h + b1, 0.0).astype(jnp.bfloat16)
        h = jnp.dot(w2, h, preferred_element_type=jnp.float32)
        h = jnp.maximum(h + b2, 0.0).astype(jnp.bfloat16)
        y = jnp.dot(w3, h, preferred_element_type=jnp.float32)
        o_ref[:, lanes] = y + b3


@jax.jit
def _forward(x, packed):
    bsz = x.shape[0]
    tb = _TB if bsz % _TB == 0 else bsz
    b_pad = pl.cdiv(bsz, tb) * tb

    x_fm = jnp.asarray(x, jnp.float32).T
    if b_pad != bsz:
        x_fm = jnp.pad(x_fm, ((0, 0), (0, b_pad - bsz)))

    out_fm = pl.pallas_call(
        _mlp_kernel,
        out_shape=jax.ShapeDtypeStruct((1, b_pad), jnp.float32),
        grid=(b_pad // tb,),
        in_specs=[
            pl.BlockSpec((32, 128), lambda i: (0, 0)),
            pl.BlockSpec((4, tb), lambda i: (0, i)),
        ],
        out_specs=pl.BlockSpec((1, tb), lambda i: (0, i)),
        compiler_params=pltpu.CompilerParams(
            dimension_semantics=("parallel",),
            vmem_limit_bytes=64 * 1024 * 1024,
        ),
    )(packed, x_fm)

    return out_fm[0, :bsz].reshape(bsz, 1)


def kernel(x, packed):
    return _forward(x, packed)


# speedup vs baseline: 33.5399x; 1.1299x over previous
import jax
import jax.numpy as jnp
from jax.experimental import pallas as pl
from jax.experimental.pallas import tpu as pltpu

_TB = 131072
_CHUNK = 8192
_NC = _TB // _CHUNK


def _mlp_kernel(p_ref, x_ref, o_ref, h2_ref):
    w1 = p_ref[0:16, 0:4].astype(jnp.bfloat16)
    b1 = p_ref[0:16, 4:5].astype(jnp.bfloat16)
    w2 = p_ref[16:32, 0:16].astype(jnp.bfloat16)
    b2 = p_ref[16:32, 16:17].astype(jnp.bfloat16)
    b3 = p_ref[0:1, 80:81]

    for c in range(_NC):
        lanes = pl.ds(c * _CHUNK, _CHUNK)
        x = x_ref[:, lanes].astype(jnp.bfloat16)
        h = jnp.dot(w1, x, preferred_element_type=jnp.float32)
        h = jnp.maximum(h.astype(jnp.bfloat16) + b1, 0)
        h = jnp.dot(w2, h, preferred_element_type=jnp.float32)
        h = jnp.maximum(h.astype(jnp.bfloat16) + b2, 0)
        h2_ref[c * 16:(c + 1) * 16, :] = h

    w3 = p_ref[0:1, 64:80].astype(jnp.bfloat16)
    w3t = jnp.tile(w3, (16, 16))
    row = jax.lax.broadcasted_iota(jnp.int32, (16, 16 * 16), 0)
    col = jax.lax.broadcasted_iota(jnp.int32, (16, 16 * 16), 1)
    w3big = jnp.where(col // 16 == row, w3t, jnp.bfloat16(0))
    y = jnp.dot(w3big, h2_ref[...], preferred_element_type=jnp.float32)
    o_ref[0, :, :] = y + b3


@jax.jit
def _forward(x, packed):
    bsz = x.shape[0]
    assert bsz % _TB == 0, "batch pinned by the pipeline"
    steps = bsz // _TB

    x_fm = jnp.asarray(x, jnp.float32).T

    out3 = pl.pallas_call(
        _mlp_kernel,
        out_shape=jax.ShapeDtypeStruct((steps, _NC, _CHUNK), jnp.float32),
        grid=(steps,),
        in_specs=[
            pl.BlockSpec((32, 128), lambda i: (0, 0)),
            pl.BlockSpec((4, _TB), lambda i: (0, i)),
        ],
        out_specs=pl.BlockSpec((1, _NC, _CHUNK), lambda i: (i, 0, 0)),
        scratch_shapes=[pltpu.VMEM((16 * _NC, _CHUNK), jnp.bfloat16)],
        compiler_params=pltpu.CompilerParams(
            dimension_semantics=("parallel",),
            vmem_limit_bytes=64 * 1024 * 1024,
        ),
    )(packed, x_fm)

    return out3.reshape(bsz, 1)


def kernel(x, packed):
    return _forward(x, packed)


# speedup vs baseline: 41.5647x; 1.2393x over previous
import jax
import jax.numpy as jnp
from jax.experimental import pallas as pl
from jax.experimental.pallas import tpu as pltpu

_TB = 131072
_CHUNK = 8192
_NC = _TB // _CHUNK


def _mlp_kernel(p_ref, x_ref, o_ref, h2_ref):
    w1 = p_ref[0:16, 0:4].astype(jnp.bfloat16)
    b1 = p_ref[0:16, 4:5].astype(jnp.bfloat16)
    w2 = p_ref[16:32, 0:16].astype(jnp.bfloat16)
    b2 = p_ref[16:32, 16:17].astype(jnp.bfloat16)
    b3 = p_ref[0:1, 80:81]

    for c in range(_NC):
        lanes = pl.ds(c * _CHUNK, _CHUNK)
        x = x_ref[:, lanes].astype(jnp.bfloat16)
        h = jnp.dot(w1, x, preferred_element_type=jnp.float32)
        h = jnp.maximum(h.astype(jnp.bfloat16) + b1, 0)
        h = jnp.dot(w2, h, preferred_element_type=jnp.float32)
        h = jnp.maximum(h.astype(jnp.bfloat16) + b2, 0)
        h2_ref[c * 16:(c + 1) * 16, :] = h

    w3 = p_ref[0:1, 64:80].astype(jnp.bfloat16)
    w3t = jnp.tile(w3, (16, 16))
    row = jax.lax.broadcasted_iota(jnp.int32, (16, 16 * 16), 0)
    col = jax.lax.broadcasted_iota(jnp.int32, (16, 16 * 16), 1)
    w3big = jnp.where(col // 16 == row, w3t, jnp.bfloat16(0))
    y = jnp.dot(w3big, h2_ref[...], preferred_element_type=jnp.float32)
    o_ref[...] = (y + b3).reshape(_TB // 128, 128)


@jax.jit
def _forward(x, packed):
    bsz = x.shape[0]
    assert bsz % _TB == 0, "batch pinned by the pipeline"
    steps = bsz // _TB

    x_fm = jnp.asarray(x, jnp.float32).T

    out3 = pl.pallas_call(
        _mlp_kernel,
        out_shape=jax.ShapeDtypeStruct((bsz // 128, 128), jnp.float32),
        grid=(steps,),
        in_specs=[
            pl.BlockSpec((32, 128), lambda i: (0, 0)),
            pl.BlockSpec((4, _TB), lambda i: (0, i)),
        ],
        out_specs=pl.BlockSpec((_TB // 128, 128), lambda i: (i, 0)),
        scratch_shapes=[pltpu.VMEM((16 * _NC, _CHUNK), jnp.bfloat16)],
        compiler_params=pltpu.CompilerParams(
            dimension_semantics=("parallel",),
            vmem_limit_bytes=64 * 1024 * 1024,
        ),
    )(packed, x_fm)

    return out3.reshape(bsz, 1)


def kernel(x, packed):
    return _forward(x, packed)


# speedup vs baseline: 60.8919x; 1.4650x over previous
import jax
import jax.numpy as jnp
from jax.experimental import pallas as pl
from jax.experimental.pallas import tpu as pltpu

_TB = 262144
_CHUNK = 16384
_NC = _TB // _CHUNK


def _mlp_kernel(p_ref, x_ref, o_ref, h2_ref):
    w1 = p_ref[0:16, 0:4]
    b1 = p_ref[0:16, 4:5].astype(jnp.bfloat16)
    w2 = p_ref[16:32, 0:16].astype(jnp.bfloat16)
    b2 = p_ref[16:32, 16:17].astype(jnp.bfloat16)
    b3 = p_ref[0:1, 80:81]

    for c in range(_NC):
        lanes = pl.ds(c * _CHUNK, _CHUNK)
        x = x_ref[:, lanes]
        h = jnp.dot(w1, x, preferred_element_type=jnp.float32)
        h = jnp.maximum(h.astype(jnp.bfloat16) + b1, 0)
        h = jnp.dot(w2, h, preferred_element_type=jnp.float32)
        h = jnp.maximum(h.astype(jnp.bfloat16) + b2, 0)
        h2_ref[c * 16:(c + 1) * 16, :] = h

    w3 = p_ref[0:1, 64:80].astype(jnp.bfloat16)
    w3t = jnp.tile(w3, (_NC, _NC))
    row = jax.lax.broadcasted_iota(jnp.int32, (_NC, 16 * _NC), 0)
    col = jax.lax.broadcasted_iota(jnp.int32, (_NC, 16 * _NC), 1)
    w3big = jnp.where(col // 16 == row, w3t, jnp.bfloat16(0))
    y = jnp.dot(w3big, h2_ref[...], preferred_element_type=jnp.float32)
    o_ref[...] = (y + b3).reshape(_TB // 128, 128)


@jax.jit
def _forward(x, packed):
    bsz = x.shape[0]
    assert bsz % _TB == 0, "batch pinned by the pipeline"
    steps = bsz // _TB

    x_fm = jnp.asarray(x, jnp.float32).T

    out3 = pl.pallas_call(
        _mlp_kernel,
        out_shape=jax.ShapeDtypeStruct((bsz // 128, 128), jnp.float32),
        grid=(steps,),
        in_specs=[
            pl.BlockSpec((32, 128), lambda i: (0, 0)),
            pl.BlockSpec((4, _TB), lambda i: (0, i)),
        ],
        out_specs=pl.BlockSpec((_TB // 128, 128), lambda i: (i, 0)),
        scratch_shapes=[pltpu.VMEM((16 * _NC, _CHUNK), jnp.bfloat16)],
        compiler_params=pltpu.CompilerParams(
            dimension_semantics=("parallel",),
            vmem_limit_bytes=64 * 1024 * 1024,
        ),
    )(packed, x_fm)

    return out3.reshape(bsz, 1)


def kernel(x, packed):
    return _forward(x, packed)
